# baseline (speedup 1.0000x reference)
import functools

import jax
import jax.numpy as jnp
from jax import lax
from jax.experimental import pallas as pl
from jax.experimental.pallas import tpu as pltpu
from jax.experimental.pallas import tpu_sc as plsc

B = 4
T = 4096
D = 1024
C = 8
LANES = 16
CHUNK = C * D
UNROLL = 8
NBUF = 4


def _make_sc_kernel(n_workers):
    t_per_w = T // n_workers
    n_j = t_per_w // C
    n_k = n_j * B
    mesh = plsc.VectorSubcoreMesh(core_axis_name="c", subcore_axis_name="s")
    nc = 2

    vmem = pltpu.VMEM((CHUNK,), jnp.float32)
    sem = pltpu.SemaphoreType.DMA

    @functools.partial(
        pl.kernel,
        mesh=mesh,
        out_type=jax.ShapeDtypeStruct((B * T * D,), jnp.float32),
        scratch_types=[vmem] * (2 * NBUF + 2) + [sem] * (2 * NBUF + 2),
    )
    def k(x_hbm, pos_hbm, out_hbm, *bufs):
        refs, sems = bufs[:2 * NBUF + 2], bufs[2 * NBUF + 2:]
        xbufs, obufs, pbufs = refs[:NBUF], refs[NBUF:2 * NBUF], refs[2 * NBUF:]
        sxs, sos, sps = sems[:NBUF], sems[NBUF:2 * NBUF], sems[2 * NBUF:]

        wid = lax.axis_index("s") * nc + lax.axis_index("c")
        t_base = wid * t_per_w

        def x_off(kk):
            j, b = kk // B, kk % B
            return (b * T + t_base + j * C) * D

        def p_off(j):
            return (t_base + j * C) * D

        def start_x(kk):
            return pltpu.async_copy(
                x_hbm.at[pl.ds(x_off(kk), CHUNK)],
                xbufs[kk % NBUF], sxs[kk % NBUF])

        def start_p(j):
            return pltpu.async_copy(
                pos_hbm.at[pl.ds(p_off(j), CHUNK)], pbufs[j % 2], sps[j % 2])

        h_x, h_p, h_out = {}, {}, {}
        for kk in range(NBUF - 1):
            h_x[kk] = start_x(kk)
        h_p[0] = start_p(0)

        for kk in range(n_k):
            j, b = kk // B, kk % B
            if kk + NBUF - 1 < n_k:
                h_x[kk + NBUF - 1] = start_x(kk + NBUF - 1)
            nxt = kk + 1
            if nxt < n_k and nxt % B == 0:
                h_p[nxt // B] = start_p(nxt // B)
            h_x[kk].wait()
            if b == 0:
                h_p[j].wait()
            if kk >= NBUF:
                h_out[kk - NBUF].wait()

            xbuf, pbuf, obuf = xbufs[kk % NBUF], pbufs[j % 2], obufs[kk % NBUF]

            def add_body(i, _, xbuf=xbuf, pbuf=pbuf, obuf=obuf):
                base = i * (LANES * UNROLL)
                for u in range(UNROLL):
                    s = pl.ds(base + u * LANES, LANES)
                    obuf[s] = xbuf[s] + pbuf[s]
                return 0

            lax.fori_loop(0, CHUNK // (LANES * UNROLL), add_body, 0)
            h_out[kk] = pltpu.async_copy(
                obuf, out_hbm.at[pl.ds(x_off(kk), CHUNK)], sos[kk % NBUF])

        for kk in range(n_k - NBUF, n_k):
            h_out[kk].wait()

    return k


def kernel(x, pos_table):
    info = plsc.get_sparse_core_info()
    n_workers = info.num_cores * info.num_subcores
    x_flat = x.reshape(-1)
    pos_flat = pos_table.reshape(-1)
    out_flat = _make_sc_kernel(n_workers)(x_flat, pos_flat)
    return out_flat.reshape(x.shape)


# speedup vs baseline: 5.0114x; 5.0114x over previous
import jax
import jax.numpy as jnp
from jax.experimental import pallas as pl
from jax.experimental.pallas import tpu as pltpu

BT = 2048


def _add_kernel(x_ref, pos_ref, o_ref):
    o_ref[...] = x_ref[...] + pos_ref[...]


def kernel(x, pos_table):
    B, T, D = x.shape
    num_t = T // BT
    grid = (num_t, B)
    return pl.pallas_call(
        _add_kernel,
        grid=grid,
        in_specs=[
            pl.BlockSpec((1, BT, D), lambda t, b: (b, t, 0)),
            pl.BlockSpec((BT, D), lambda t, b: (t, 0)),
        ],
        out_specs=pl.BlockSpec((1, BT, D), lambda t, b: (b, t, 0)),
        out_shape=jax.ShapeDtypeStruct((B, T, D), x.dtype),
        compiler_params=pltpu.CompilerParams(
            dimension_semantics=("arbitrary", "arbitrary"),
            vmem_limit_bytes=128 * 1024 * 1024,
        ),
    )(x, pos_table)
